# Initial kernel scaffold; baseline (speedup 1.0000x reference)
import jax
import jax.numpy as jnp
from jax.experimental import pallas as pl


def kernel(teacher_feats, student_feats, ref_perm, shared_perm):
    raise NotImplementedError("write your pallas kernel here")


# speedup vs baseline: 6.8771x; 6.8771x over previous
import functools

import jax
import jax.numpy as jnp
from jax import lax
from jax.experimental import pallas as pl
from jax.experimental.pallas import tpu as pltpu
from jax.experimental.pallas import tpu_sc as plsc

_HI = lax.Precision.HIGHEST
_B, _P, _D = 2, 1024, 1024
_NR, _NS, _K = 128, 128, 4
_PAIRS = ((2, 1), (4, 2), (6, 3))
_TOTAL = 3 * _B * _NR * _NS * _K


def _dotT(a, b):
    return lax.dot_general(a, b, (((1,), (1,)), ((), ())),
                           preferred_element_type=jnp.float32, precision=_HI)


def _sc_gather_rows(table, idx, n_rows):
    n_workers = 32
    per_w = n_rows // n_workers
    mesh = plsc.VectorSubcoreMesh(core_axis_name="c", subcore_axis_name="s")

    @functools.partial(
        pl.kernel,
        mesh=mesh,
        out_type=jax.ShapeDtypeStruct((n_rows, _D), jnp.float32),
        scratch_types=[
            pltpu.VMEM((per_w,), jnp.int32),
            pltpu.VMEM((per_w, _D), jnp.float32),
            pltpu.SemaphoreType.DMA,
        ],
    )
    def k(table_hbm, idx_hbm, out_hbm, idx_v, rows_v, sem):
        wid = lax.axis_index("s") * 2 + lax.axis_index("c")
        base = wid * per_w
        pltpu.sync_copy(idx_hbm.at[pl.ds(base, per_w)], idx_v)
        pltpu.async_copy(table_hbm.at[idx_v], rows_v, sem).wait()
        pltpu.sync_copy(rows_v, out_hbm.at[pl.ds(base, per_w)])

    return k(table, idx)


def _sim_topk_body(reft_ref, frame_ref, gidx_ref, sim_ref):
    b = pl.program_id(0)
    f = pl.program_id(1)
    frame = frame_ref[0, 0]
    reft = reft_ref[...]
    nr = jnp.sqrt(jnp.sum(reft * reft, axis=1, keepdims=True))
    rtn = reft / jnp.maximum(nr, 1e-12)
    s = _dotT(frame, rtn)
    ne = jnp.sqrt(jnp.sum(frame * frame, axis=1, keepdims=True))
    sim_ref[f] = s / jnp.maximum(ne, 1e-12)

    @pl.when(f == 3)
    def _():
        work = sim_ref[...].reshape(4 * _P, _NR)
        row_iota = lax.broadcasted_iota(jnp.int32, (4 * _P, _NR), 0)
        gs = []
        for _k in range(_K):
            m = jnp.max(work, axis=0, keepdims=True)
            idx = jnp.min(jnp.where(work == m, row_iota, 4 * _P),
                          axis=0, keepdims=True)
            fi = idx // _P
            p = idx - fi * _P
            gs.append(b * (8 * _P) + fi * (2 * _P) + _P + p)
            work = jnp.where(row_iota == idx, -jnp.inf, work)
        gidx_ref[0] = jnp.concatenate(gs, axis=0)


def _sim_topk(gath_t, teacher):
    return pl.pallas_call(
        _sim_topk_body,
        grid=(_B, 4),
        in_specs=[
            pl.BlockSpec((_NR, _D), lambda b, f: (4 * b, 0)),
            pl.BlockSpec((1, 1, _P, _D), lambda b, f: (b, 2 * f + 1, 0, 0)),
        ],
        out_specs=pl.BlockSpec((1, _K, _NR), lambda b, f: (b, 0, 0)),
        out_shape=jax.ShapeDtypeStruct((_B, _K, _NR), jnp.int32),
        scratch_shapes=[pltpu.VMEM((4, _P, _NR), jnp.float32)],
    )(gath_t, teacher)


def _angle_body(reft_ref, refs_ref, sht_ref, shs_ref, high_ref, out_ref):
    b = pl.program_id(0)
    j = pl.program_id(1)
    k = pl.program_id(2)
    high = high_ref[...]
    nh2 = jnp.sum(high * high, axis=1, keepdims=True)

    coss = []
    for refr, shr in ((reft_ref, sht_ref), (refs_ref, shs_ref)):
        refX = refr[...]
        shX = shr[...]
        ns2 = jnp.sum(shX * shX, axis=1)
        nr2 = jnp.sum(refX * refX, axis=1, keepdims=True)
        G_rs = _dotT(refX, shX)
        G_sh = _dotT(high, shX)
        G_rh = jnp.sum(refX * high, axis=1, keepdims=True)

        d_sr2 = ns2[None, :] - 2.0 * G_rs + nr2
        d_hr2 = nh2 - 2.0 * G_rh + nr2
        d_sh2 = ns2[None, :] - 2.0 * G_sh + nh2
        inv_sr = 1.0 / jnp.maximum(jnp.sqrt(jnp.maximum(d_sr2, 0.0)), 1e-8)
        inv_hr = 1.0 / jnp.maximum(jnp.sqrt(jnp.maximum(d_hr2, 0.0)), 1e-8)
        inv_sh = 1.0 / jnp.maximum(jnp.sqrt(jnp.maximum(d_sh2, 0.0)), 1e-8)

        c1 = (G_sh - G_rs - G_rh + nr2) * inv_sr * inv_hr
        c2 = (G_rs - G_rh - G_sh + nh2) * inv_hr * inv_sh
        c3 = (G_rh - G_sh - G_rs + ns2[None, :]) * inv_sr * inv_sh
        coss.append((c1, c2, c3))

    contrib = jnp.zeros((1, 1), jnp.float32)
    for a in range(3):
        contrib = contrib + jnp.sum(jnp.abs(coss[1][a] - coss[0][a]),
                                    axis=(0, 1), keepdims=True)

    @pl.when((b == 0) & (j == 0) & (k == 0))
    def _():
        out_ref[...] = jnp.zeros((1, 1), jnp.float32)

    out_ref[...] += contrib

    @pl.when((b == _B - 1) & (j == 2) & (k == _K - 1))
    def _():
        out_ref[...] = out_ref[...] / jnp.float32(_TOTAL)


def _angle_loss(gath_t, gath_s, high):
    return pl.pallas_call(
        _angle_body,
        grid=(_B, 3, _K),
        in_specs=[
            pl.BlockSpec((_NR, _D), lambda b, j, k: (4 * b, 0)),
            pl.BlockSpec((_NR, _D), lambda b, j, k: (4 * b, 0)),
            pl.BlockSpec((_NR, _D), lambda b, j, k: (4 * b + 1 + j, 0)),
            pl.BlockSpec((_NR, _D), lambda b, j, k: (4 * b + 1 + j, 0)),
            pl.BlockSpec((_NR, _D), lambda b, j, k: (4 * b + k, 0)),
        ],
        out_specs=pl.BlockSpec((1, 1), lambda b, j, k: (0, 0)),
        out_shape=jax.ShapeDtypeStruct((1, 1), jnp.float32),
    )(gath_t, gath_s, gath_t, gath_s, high)


def kernel(teacher_feats, student_feats, ref_perm, shared_perm):
    tf = teacher_feats.reshape(_B * 8 * _P, _D)
    sf = student_feats.reshape(_B * 4 * _P, _D)
    rp = ref_perm.astype(jnp.int32)
    sp = shared_perm.astype(jnp.int32)

    t_rows, s_rows = [], []
    for b in range(_B):
        t_rows.append(b * (8 * _P) + rp)
        s_rows.append(b * (4 * _P) + rp)
        for tfi, sfi in _PAIRS:
            t_rows.append(b * (8 * _P) + tfi * _P + sp)
            s_rows.append(b * (4 * _P) + sfi * _P + sp)
    idx_t = jnp.concatenate(t_rows)
    idx_s = jnp.concatenate(s_rows)

    gath_t = _sc_gather_rows(tf, idx_t, 1024)
    gath_s = _sc_gather_rows(sf, idx_s, 1024)
    gidx = _sim_topk(gath_t, teacher_feats)
    high = _sc_gather_rows(tf, gidx.reshape(1024), 1024)
    loss = _angle_loss(gath_t, gath_s, high)
    return loss.reshape(())


# speedup vs baseline: 28.2388x; 4.1062x over previous
import functools

import jax
import jax.numpy as jnp
from jax import lax
from jax.experimental import pallas as pl
from jax.experimental.pallas import tpu as pltpu
from jax.experimental.pallas import tpu_sc as plsc

_HI = lax.Precision.HIGHEST
_B, _P, _D = 2, 1024, 1024
_NR, _NS, _K = 128, 128, 4
_PAIRS = ((2, 1), (4, 2), (6, 3))
_TOTAL = 3 * _B * _NR * _NS * _K


def _dotT(a, b):
    return lax.dot_general(a, b, (((1,), (1,)), ((), ())),
                           preferred_element_type=jnp.float32, precision=_HI)


def _sc_gather_rows(table, idx, n_rows):
    n_workers = 32
    per_w = n_rows // n_workers
    mesh = plsc.VectorSubcoreMesh(core_axis_name="c", subcore_axis_name="s")

    @functools.partial(
        pl.kernel,
        mesh=mesh,
        out_type=jax.ShapeDtypeStruct((n_rows, _D), jnp.float32),
        scratch_types=[
            pltpu.VMEM((per_w,), jnp.int32),
            pltpu.VMEM((per_w, _D), jnp.float32),
            pltpu.SemaphoreType.DMA,
        ],
    )
    def k(table_hbm, idx_hbm, out_hbm, idx_v, rows_v, sem):
        wid = lax.axis_index("s") * 2 + lax.axis_index("c")
        base = wid * per_w
        pltpu.sync_copy(idx_hbm.at[pl.ds(base, per_w)], idx_v)
        pltpu.async_copy(table_hbm.at[idx_v], rows_v, sem).wait()
        pltpu.sync_copy(rows_v, out_hbm.at[pl.ds(base, per_w)])

    return k(table, idx)


def _sim_topk_body(reft_ref, frame_ref, gidx_ref, sim_ref):
    b = pl.program_id(0)
    f = pl.program_id(1)
    frame = frame_ref[0, 0]
    reft = reft_ref[...]
    nr = jnp.sqrt(jnp.sum(reft * reft, axis=1, keepdims=True))
    rtn = reft / jnp.maximum(nr, 1e-12)
    s = _dotT(frame, rtn)
    ne = jnp.sqrt(jnp.sum(frame * frame, axis=1, keepdims=True))
    sim_ref[f] = s / jnp.maximum(ne, 1e-12)

    @pl.when(f == 3)
    def _():
        work = sim_ref[...].reshape(4 * _P, _NR)
        row_iota = lax.broadcasted_iota(jnp.int32, (4 * _P, _NR), 0)
        gs = []
        for _k in range(_K):
            m = jnp.max(work, axis=0, keepdims=True)
            idx = jnp.min(jnp.where(work == m, row_iota, 4 * _P),
                          axis=0, keepdims=True)
            fi = idx // _P
            p = idx - fi * _P
            gs.append(b * (8 * _P) + fi * (2 * _P) + _P + p)
            work = jnp.where(row_iota == idx, -jnp.inf, work)
        gidx_ref[0] = jnp.concatenate(gs, axis=0)


def _sim_topk(gath_t, teacher):
    return pl.pallas_call(
        _sim_topk_body,
        grid=(_B, 4),
        in_specs=[
            pl.BlockSpec((_NR, _D), lambda b, f: (4 * b, 0)),
            pl.BlockSpec((1, 1, _P, _D), lambda b, f: (b, 2 * f + 1, 0, 0)),
        ],
        out_specs=pl.BlockSpec((1, _K, _NR), lambda b, f: (b, 0, 0)),
        out_shape=jax.ShapeDtypeStruct((_B, _K, _NR), jnp.int32),
        scratch_shapes=[pltpu.VMEM((4, _P, _NR), jnp.float32)],
    )(gath_t, teacher)


def _angle_body(reft_ref, refs_ref, sht_ref, shs_ref, high_ref, out_ref):
    b = pl.program_id(0)
    j = pl.program_id(1)
    k = pl.program_id(2)
    high = high_ref[...]
    nh2 = jnp.sum(high * high, axis=1, keepdims=True)
    ones_row = jnp.ones((1, _D), jnp.float32)

    def _inv_clamped(d2):
        return jnp.where(d2 > 1e-16, lax.rsqrt(jnp.maximum(d2, 1e-30)),
                         jnp.float32(1e8))

    coss = []
    for refr, shr in ((reft_ref, sht_ref), (refs_ref, shs_ref)):
        refX = refr[...]
        shX = shr[...]
        ns2r = _dotT(ones_row, shX * shX)
        nr2 = jnp.sum(refX * refX, axis=1, keepdims=True)
        G_rs = _dotT(refX, shX)
        G_sh = _dotT(high, shX)
        G_rh = jnp.sum(refX * high, axis=1, keepdims=True)

        d_sr2 = ns2r - 2.0 * G_rs + nr2
        d_hr2 = nh2 - 2.0 * G_rh + nr2
        d_sh2 = ns2r - 2.0 * G_sh + nh2
        inv_sr = _inv_clamped(d_sr2)
        inv_hr = _inv_clamped(d_hr2)
        inv_sh = _inv_clamped(d_sh2)

        c1 = (G_sh - G_rs - G_rh + nr2) * inv_sr * inv_hr
        c2 = (G_rs - G_rh - G_sh + nh2) * inv_hr * inv_sh
        c3 = (G_rh - G_sh - G_rs + ns2r) * inv_sr * inv_sh
        coss.append((c1, c2, c3))

    contrib = jnp.zeros((1, 1), jnp.float32)
    for a in range(3):
        contrib = contrib + jnp.sum(jnp.abs(coss[1][a] - coss[0][a]),
                                    axis=(0, 1), keepdims=True)

    @pl.when((b == 0) & (j == 0) & (k == 0))
    def _():
        out_ref[...] = jnp.zeros((1, 1), jnp.float32)

    out_ref[...] += contrib

    @pl.when((b == _B - 1) & (j == 2) & (k == _K - 1))
    def _():
        out_ref[...] = out_ref[...] / jnp.float32(_TOTAL)


def _angle_loss(gath_t, gath_s, high):
    return pl.pallas_call(
        _angle_body,
        grid=(_B, 3, _K),
        in_specs=[
            pl.BlockSpec((_NR, _D), lambda b, j, k: (4 * b, 0)),
            pl.BlockSpec((_NR, _D), lambda b, j, k: (4 * b, 0)),
            pl.BlockSpec((_NR, _D), lambda b, j, k: (4 * b + 1 + j, 0)),
            pl.BlockSpec((_NR, _D), lambda b, j, k: (4 * b + 1 + j, 0)),
            pl.BlockSpec((_NR, _D), lambda b, j, k: (4 * b + k, 0)),
        ],
        out_specs=pl.BlockSpec((1, 1), lambda b, j, k: (0, 0)),
        out_shape=jax.ShapeDtypeStruct((1, 1), jnp.float32),
    )(gath_t, gath_s, gath_t, gath_s, high)


def kernel(teacher_feats, student_feats, ref_perm, shared_perm):
    tf = teacher_feats.reshape(_B * 8 * _P, _D)
    sf = student_feats.reshape(_B * 4 * _P, _D)
    rp = ref_perm.astype(jnp.int32)
    sp = shared_perm.astype(jnp.int32)

    t_rows, s_rows = [], []
    for b in range(_B):
        t_rows.append(b * (8 * _P) + rp)
        s_rows.append(b * (4 * _P) + rp)
        for tfi, sfi in _PAIRS:
            t_rows.append(b * (8 * _P) + tfi * _P + sp)
            s_rows.append(b * (4 * _P) + sfi * _P + sp)
    idx_t = jnp.concatenate(t_rows)
    idx_s = jnp.concatenate(s_rows)

    gath_t = _sc_gather_rows(tf, idx_t, 1024)
    gath_s = _sc_gather_rows(sf, idx_s, 1024)
    gidx = _sim_topk(gath_t, teacher_feats)
    high = _sc_gather_rows(tf, gidx.reshape(1024), 1024)
    loss = _angle_loss(gath_t, gath_s, high)
    return loss.reshape(())


# speedup vs baseline: 37.2404x; 1.3188x over previous
import functools

import jax
import jax.numpy as jnp
from jax import lax
from jax.experimental import pallas as pl
from jax.experimental.pallas import tpu as pltpu
from jax.experimental.pallas import tpu_sc as plsc

_HI = lax.Precision.HIGHEST
_B, _P, _D = 2, 1024, 1024
_NR, _NS, _K = 128, 128, 4
_PAIRS = ((2, 1), (4, 2), (6, 3))
_TOTAL = 3 * _B * _NR * _NS * _K


def _dotT(a, b, precision=_HI):
    return lax.dot_general(a, b, (((1,), (1,)), ((), ())),
                           preferred_element_type=jnp.float32,
                           precision=precision)


def _sc_gather_rows(table, idx, n_rows):
    n_workers = 32
    per_w = n_rows // n_workers
    mesh = plsc.VectorSubcoreMesh(core_axis_name="c", subcore_axis_name="s")

    @functools.partial(
        pl.kernel,
        mesh=mesh,
        out_type=jax.ShapeDtypeStruct((n_rows, _D), jnp.float32),
        scratch_types=[
            pltpu.VMEM((per_w,), jnp.int32),
            pltpu.VMEM((per_w, _D), jnp.float32),
            pltpu.SemaphoreType.DMA,
        ],
    )
    def k(table_hbm, idx_hbm, out_hbm, idx_v, rows_v, sem):
        wid = lax.axis_index("s") * 2 + lax.axis_index("c")
        base = wid * per_w
        pltpu.sync_copy(idx_hbm.at[pl.ds(base, per_w)], idx_v)
        pltpu.async_copy(table_hbm.at[idx_v], rows_v, sem).wait()
        pltpu.sync_copy(rows_v, out_hbm.at[pl.ds(base, per_w)])

    return k(table, idx)


def _sim_topk_body(reft_ref, frame_ref, gidx_ref, sim_ref):
    b = pl.program_id(0)
    f = pl.program_id(1)
    frame = frame_ref[0, 0]
    reft = reft_ref[...]
    nr2 = jnp.sum(reft * reft, axis=1, keepdims=True)
    rtn = reft * jnp.where(nr2 > 1e-24, lax.rsqrt(jnp.maximum(nr2, 1e-30)),
                           jnp.float32(1e12))
    s = _dotT(frame, rtn)
    ne2 = jnp.sum(frame * frame, axis=1, keepdims=True)
    sim_ref[f] = s * jnp.where(ne2 > 1e-24, lax.rsqrt(jnp.maximum(ne2, 1e-30)),
                               jnp.float32(1e12))

    @pl.when(f == 3)
    def _():
        work = sim_ref[...].reshape(4 * _P, _NR)
        row_iota = lax.broadcasted_iota(jnp.int32, (4 * _P, _NR), 0)
        gs = []
        for _k in range(_K):
            m = jnp.max(work, axis=0, keepdims=True)
            idx = jnp.min(jnp.where(work == m, row_iota, 4 * _P),
                          axis=0, keepdims=True)
            fi = idx // _P
            p = idx - fi * _P
            gs.append(b * (8 * _P) + fi * (2 * _P) + _P + p)
            work = jnp.where(row_iota == idx, -jnp.inf, work)
        gidx_ref[0] = jnp.concatenate(gs, axis=0)


def _sim_topk(gath_t, teacher):
    return pl.pallas_call(
        _sim_topk_body,
        grid=(_B, 4),
        in_specs=[
            pl.BlockSpec((_NR, _D), lambda b, f: (4 * b, 0)),
            pl.BlockSpec((1, 1, _P, _D), lambda b, f: (b, 2 * f + 1, 0, 0)),
        ],
        out_specs=pl.BlockSpec((1, _K, _NR), lambda b, f: (b, 0, 0)),
        out_shape=jax.ShapeDtypeStruct((_B, _K, _NR), jnp.int32),
        scratch_shapes=[pltpu.VMEM((4, _P, _NR), jnp.float32)],
    )(gath_t, teacher)


def _angle_body(reft_ref, refs_ref, sht_ref, shs_ref, high_ref, out_ref):
    b = pl.program_id(0)
    j = pl.program_id(1)
    k = pl.program_id(2)
    high = high_ref[...]
    nh2 = jnp.sum(high * high, axis=1, keepdims=True)
    ones_row = jnp.ones((1, _D), jnp.float32)

    def _inv_clamped(d2):
        return jnp.where(d2 > 1e-16, lax.rsqrt(jnp.maximum(d2, 1e-30)),
                         jnp.float32(1e8))

    coss = []
    for refr, shr in ((reft_ref, sht_ref), (refs_ref, shs_ref)):
        refX = refr[...]
        shX = shr[...]
        ns2r = _dotT(ones_row, shX * shX)
        nr2 = jnp.sum(refX * refX, axis=1, keepdims=True)
        G_rs = _dotT(refX, shX, lax.Precision.DEFAULT)
        G_sh = _dotT(high, shX, lax.Precision.DEFAULT)
        G_rh = jnp.sum(refX * high, axis=1, keepdims=True)

        d_sr2 = ns2r - 2.0 * G_rs + nr2
        d_hr2 = nh2 - 2.0 * G_rh + nr2
        d_sh2 = ns2r - 2.0 * G_sh + nh2
        inv_sr = _inv_clamped(d_sr2)
        inv_hr = _inv_clamped(d_hr2)
        inv_sh = _inv_clamped(d_sh2)

        c1 = (G_sh - G_rs - G_rh + nr2) * inv_sr * inv_hr
        c2 = (G_rs - G_rh - G_sh + nh2) * inv_hr * inv_sh
        c3 = (G_rh - G_sh - G_rs + ns2r) * inv_sr * inv_sh
        coss.append((c1, c2, c3))

    contrib = jnp.zeros((1, 1), jnp.float32)
    for a in range(3):
        contrib = contrib + jnp.sum(jnp.abs(coss[1][a] - coss[0][a]),
                                    axis=(0, 1), keepdims=True)

    @pl.when((b == 0) & (j == 0) & (k == 0))
    def _():
        out_ref[...] = jnp.zeros((1, 1), jnp.float32)

    out_ref[...] += contrib

    @pl.when((b == _B - 1) & (j == 2) & (k == _K - 1))
    def _():
        out_ref[...] = out_ref[...] / jnp.float32(_TOTAL)


def _angle_loss(gath_t, gath_s, high):
    return pl.pallas_call(
        _angle_body,
        grid=(_B, 3, _K),
        in_specs=[
            pl.BlockSpec((_NR, _D), lambda b, j, k: (4 * b, 0)),
            pl.BlockSpec((_NR, _D), lambda b, j, k: (4 * b, 0)),
            pl.BlockSpec((_NR, _D), lambda b, j, k: (4 * b + 1 + j, 0)),
            pl.BlockSpec((_NR, _D), lambda b, j, k: (4 * b + 1 + j, 0)),
            pl.BlockSpec((_NR, _D), lambda b, j, k: (4 * b + k, 0)),
        ],
        out_specs=pl.BlockSpec((1, 1), lambda b, j, k: (0, 0)),
        out_shape=jax.ShapeDtypeStruct((1, 1), jnp.float32),
    )(gath_t, gath_s, gath_t, gath_s, high)


def kernel(teacher_feats, student_feats, ref_perm, shared_perm):
    tf = teacher_feats.reshape(_B * 8 * _P, _D)
    sf = student_feats.reshape(_B * 4 * _P, _D)
    rp = ref_perm.astype(jnp.int32)
    sp = shared_perm.astype(jnp.int32)

    t_rows, s_rows = [], []
    for b in range(_B):
        t_rows.append(b * (8 * _P) + rp)
        s_rows.append(b * (4 * _P) + rp)
        for tfi, sfi in _PAIRS:
            t_rows.append(b * (8 * _P) + tfi * _P + sp)
            s_rows.append(b * (4 * _P) + sfi * _P + sp)
    idx_t = jnp.concatenate(t_rows)
    idx_s = jnp.concatenate(s_rows)

    gath_t = _sc_gather_rows(tf, idx_t, 1024)
    gath_s = _sc_gather_rows(sf, idx_s, 1024)
    gidx = _sim_topk(gath_t, teacher_feats)
    high = _sc_gather_rows(tf, gidx.reshape(1024), 1024)
    loss = _angle_loss(gath_t, gath_s, high)
    return loss.reshape(())


# speedup vs baseline: 43.8429x; 1.1773x over previous
import functools

import jax
import jax.numpy as jnp
from jax import lax
from jax.experimental import pallas as pl
from jax.experimental.pallas import tpu as pltpu
from jax.experimental.pallas import tpu_sc as plsc

_HI = lax.Precision.HIGHEST
_B, _P, _D = 2, 1024, 1024
_NR, _NS, _K = 128, 128, 4
_PAIRS = ((2, 1), (4, 2), (6, 3))
_TOTAL = 3 * _B * _NR * _NS * _K


def _dotT(a, b, precision=_HI):
    return lax.dot_general(a, b, (((1,), (1,)), ((), ())),
                           preferred_element_type=jnp.float32,
                           precision=precision)


def _sc_gather_rows(table, idx, n_rows):
    n_workers = 32
    per_w = n_rows // n_workers
    mesh = plsc.VectorSubcoreMesh(core_axis_name="c", subcore_axis_name="s")

    @functools.partial(
        pl.kernel,
        mesh=mesh,
        out_type=jax.ShapeDtypeStruct((n_rows, _D), jnp.float32),
        scratch_types=[
            pltpu.VMEM((per_w,), jnp.int32),
            pltpu.VMEM((per_w, _D), jnp.float32),
            pltpu.SemaphoreType.DMA,
        ],
    )
    def k(table_hbm, idx_hbm, out_hbm, idx_v, rows_v, sem):
        wid = lax.axis_index("s") * 2 + lax.axis_index("c")
        base = wid * per_w
        pltpu.sync_copy(idx_hbm.at[pl.ds(base, per_w)], idx_v)
        pltpu.async_copy(table_hbm.at[idx_v], rows_v, sem).wait()
        pltpu.sync_copy(rows_v, out_hbm.at[pl.ds(base, per_w)])

    return k(table, idx)


def _sim_topk_body(reft_ref, frame_ref, gidx_ref, sim_ref):
    b = pl.program_id(0)
    f = pl.program_id(1)
    frame = frame_ref[0, 0]
    reft = reft_ref[...]
    nr2 = jnp.sum(reft * reft, axis=1, keepdims=True)
    rtn = reft * jnp.where(nr2 > 1e-24, lax.rsqrt(jnp.maximum(nr2, 1e-30)),
                           jnp.float32(1e12))
    s = _dotT(frame, rtn, lax.Precision.DEFAULT)
    ne2 = jnp.sum(frame * frame, axis=1, keepdims=True)
    sim_ref[f] = s * jnp.where(ne2 > 1e-24, lax.rsqrt(jnp.maximum(ne2, 1e-30)),
                               jnp.float32(1e12))

    @pl.when(f == 3)
    def _():
        work = sim_ref[...].reshape(4 * _P, _NR)
        row_iota = lax.broadcasted_iota(jnp.int32, (4 * _P, _NR), 0)
        gs = []
        for _k in range(_K):
            m = jnp.max(work, axis=0, keepdims=True)
            idx = jnp.min(jnp.where(work == m, row_iota, 4 * _P),
                          axis=0, keepdims=True)
            fi = idx // _P
            p = idx - fi * _P
            gs.append(b * (8 * _P) + fi * (2 * _P) + _P + p)
            work = jnp.where(row_iota == idx, -jnp.inf, work)
        gidx_ref[0] = jnp.concatenate(gs, axis=0)


def _sim_topk(gath_t, teacher):
    return pl.pallas_call(
        _sim_topk_body,
        grid=(_B, 4),
        in_specs=[
            pl.BlockSpec((_NR, _D), lambda b, f: (4 * b, 0)),
            pl.BlockSpec((1, 1, _P, _D), lambda b, f: (b, 2 * f + 1, 0, 0)),
        ],
        out_specs=pl.BlockSpec((1, _K, _NR), lambda b, f: (b, 0, 0)),
        out_shape=jax.ShapeDtypeStruct((_B, _K, _NR), jnp.int32),
        scratch_shapes=[pltpu.VMEM((4, _P, _NR), jnp.float32)],
    )(gath_t, teacher)


def _angle_body(reft_ref, refs_ref, sht_ref, shs_ref, high_ref, out_ref):
    b = pl.program_id(0)
    j = pl.program_id(1)
    k = pl.program_id(2)
    high = high_ref[...]
    nh2 = jnp.sum(high * high, axis=1, keepdims=True)
    ones_row = jnp.ones((1, _D), jnp.float32)

    def _inv_clamped(d2):
        return jnp.where(d2 > 1e-16, lax.rsqrt(jnp.maximum(d2, 1e-30)),
                         jnp.float32(1e8))

    coss = []
    for refr, shr in ((reft_ref, sht_ref), (refs_ref, shs_ref)):
        refX = refr[...]
        shX = shr[...]
        ns2r = _dotT(ones_row, shX * shX)
        nr2 = jnp.sum(refX * refX, axis=1, keepdims=True)
        G_rs = _dotT(refX, shX, lax.Precision.DEFAULT)
        G_sh = _dotT(high, shX, lax.Precision.DEFAULT)
        G_rh = jnp.sum(refX * high, axis=1, keepdims=True)

        d_sr2 = ns2r - 2.0 * G_rs + nr2
        d_hr2 = nh2 - 2.0 * G_rh + nr2
        d_sh2 = ns2r - 2.0 * G_sh + nh2
        inv_sr = _inv_clamped(d_sr2)
        inv_hr = _inv_clamped(d_hr2)
        inv_sh = _inv_clamped(d_sh2)

        c1 = (G_sh - G_rs - G_rh + nr2) * inv_sr * inv_hr
        c2 = (G_rs - G_rh - G_sh + nh2) * inv_hr * inv_sh
        c3 = (G_rh - G_sh - G_rs + ns2r) * inv_sr * inv_sh
        coss.append((c1, c2, c3))

    contrib = jnp.zeros((1, 1), jnp.float32)
    for a in range(3):
        contrib = contrib + jnp.sum(jnp.abs(coss[1][a] - coss[0][a]),
                                    axis=(0, 1), keepdims=True)

    @pl.when((b == 0) & (j == 0) & (k == 0))
    def _():
        out_ref[...] = jnp.zeros((1, 1), jnp.float32)

    out_ref[...] += contrib

    @pl.when((b == _B - 1) & (j == 2) & (k == _K - 1))
    def _():
        out_ref[...] = out_ref[...] / jnp.float32(_TOTAL)


def _angle_loss(gath_t, gath_s, high):
    return pl.pallas_call(
        _angle_body,
        grid=(_B, 3, _K),
        in_specs=[
            pl.BlockSpec((_NR, _D), lambda b, j, k: (4 * b, 0)),
            pl.BlockSpec((_NR, _D), lambda b, j, k: (4 * b, 0)),
            pl.BlockSpec((_NR, _D), lambda b, j, k: (4 * b + 1 + j, 0)),
            pl.BlockSpec((_NR, _D), lambda b, j, k: (4 * b + 1 + j, 0)),
            pl.BlockSpec((_NR, _D), lambda b, j, k: (4 * b + k, 0)),
        ],
        out_specs=pl.BlockSpec((1, 1), lambda b, j, k: (0, 0)),
        out_shape=jax.ShapeDtypeStruct((1, 1), jnp.float32),
    )(gath_t, gath_s, gath_t, gath_s, high)


def kernel(teacher_feats, student_feats, ref_perm, shared_perm):
    tf = teacher_feats.reshape(_B * 8 * _P, _D)
    sf = student_feats.reshape(_B * 4 * _P, _D)
    rp = ref_perm.astype(jnp.int32)
    sp = shared_perm.astype(jnp.int32)

    t_rows, s_rows = [], []
    for b in range(_B):
        t_rows.append(b * (8 * _P) + rp)
        s_rows.append(b * (4 * _P) + rp)
        for tfi, sfi in _PAIRS:
            t_rows.append(b * (8 * _P) + tfi * _P + sp)
            s_rows.append(b * (4 * _P) + sfi * _P + sp)
    idx_t = jnp.concatenate(t_rows)
    idx_s = jnp.concatenate(s_rows)

    gath_t = _sc_gather_rows(tf, idx_t, 1024)
    gath_s = _sc_gather_rows(sf, idx_s, 1024)
    gidx = _sim_topk(gath_t, teacher_feats)
    high = _sc_gather_rows(tf, gidx.reshape(1024), 1024)
    loss = _angle_loss(gath_t, gath_s, high)
    return loss.reshape(())


# speedup vs baseline: 57.5634x; 1.3129x over previous
import functools

import jax
import jax.numpy as jnp
from jax import lax
from jax.experimental import pallas as pl
from jax.experimental.pallas import tpu as pltpu
from jax.experimental.pallas import tpu_sc as plsc

_B, _P, _D = 2, 1024, 1024
_NR, _NS, _K = 128, 128, 4
_PAIRS = ((2, 1), (4, 2), (6, 3))
_TOTAL = 3 * _B * _NR * _NS * _K


def _dotT(a, b):
    return lax.dot_general(a, b, (((1,), (1,)), ((), ())),
                           preferred_element_type=jnp.float32,
                           precision=lax.Precision.DEFAULT)


def _sc_gather_rows(table, idx, n_rows):
    n_workers = 32
    per_w = n_rows // n_workers
    mesh = plsc.VectorSubcoreMesh(core_axis_name="c", subcore_axis_name="s")

    @functools.partial(
        pl.kernel,
        mesh=mesh,
        out_type=jax.ShapeDtypeStruct((n_rows, _D), jnp.float32),
        scratch_types=[
            pltpu.VMEM((per_w,), jnp.int32),
            pltpu.VMEM((per_w, _D), jnp.float32),
            pltpu.SemaphoreType.DMA,
        ],
    )
    def k(table_hbm, idx_hbm, out_hbm, idx_v, rows_v, sem):
        wid = lax.axis_index("s") * 2 + lax.axis_index("c")
        base = wid * per_w
        pltpu.sync_copy(idx_hbm.at[pl.ds(base, per_w)], idx_v)
        pltpu.async_copy(table_hbm.at[idx_v], rows_v, sem).wait()
        pltpu.sync_copy(rows_v, out_hbm.at[pl.ds(base, per_w)])

    return k(table, idx)


def _sim_topk_body(reft_ref, frame_ref, gidx_ref, sim_ref):
    b = pl.program_id(0)
    f = pl.program_id(1)
    frame = frame_ref[0, 0]
    reft = reft_ref[...]
    nr2 = jnp.sum(reft * reft, axis=1, keepdims=True)
    rtn = reft * jnp.where(nr2 > 1e-24, lax.rsqrt(jnp.maximum(nr2, 1e-30)),
                           jnp.float32(1e12))
    s = _dotT(frame, rtn)
    ne2 = jnp.sum(frame * frame, axis=1, keepdims=True)
    sim_ref[f] = s * jnp.where(ne2 > 1e-24, lax.rsqrt(jnp.maximum(ne2, 1e-30)),
                               jnp.float32(1e12))

    @pl.when(f == 3)
    def _():
        work = sim_ref[...].reshape(4 * _P, _NR)
        row_iota = lax.broadcasted_iota(jnp.int32, (4 * _P, _NR), 0)
        gs = []
        for _k in range(_K):
            m = jnp.max(work, axis=0, keepdims=True)
            idx = jnp.min(jnp.where(work == m, row_iota, 4 * _P),
                          axis=0, keepdims=True)
            fi = idx // _P
            p = idx - fi * _P
            gs.append(b * (8 * _P) + fi * (2 * _P) + _P + p)
            work = jnp.where(row_iota == idx, -jnp.inf, work)
        gidx_ref[0] = jnp.concatenate(gs, axis=0)


def _sim_topk(gath_t, teacher):
    return pl.pallas_call(
        _sim_topk_body,
        grid=(_B, 4),
        in_specs=[
            pl.BlockSpec((_NR, _D), lambda b, f: (4 * b, 0)),
            pl.BlockSpec((1, 1, _P, _D), lambda b, f: (b, 2 * f + 1, 0, 0)),
        ],
        out_specs=pl.BlockSpec((1, _K, _NR), lambda b, f: (b, 0, 0)),
        out_shape=jax.ShapeDtypeStruct((_B, _K, _NR), jnp.int32),
        scratch_shapes=[pltpu.VMEM((4, _P, _NR), jnp.float32)],
    )(gath_t, teacher)


def _angle_body(reft_ref, refs_ref, sht_ref, shs_ref, high_ref, out_ref):
    b = pl.program_id(0)
    j = pl.program_id(1)
    high = high_ref[...]
    ones_row = jnp.ones((1, _D), jnp.float32)

    def _inv_clamped(d2):
        return jnp.where(d2 > 1e-16, lax.rsqrt(jnp.maximum(d2, 1e-30)),
                         jnp.float32(1e8))

    nh2 = _dotT(high * high, ones_row)
    rows = lax.broadcasted_iota(jnp.int32, (4 * _NR, _NR), 0)
    cols = lax.broadcasted_iota(jnp.int32, (4 * _NR, _NR), 1)
    diag = rows % _NR == cols

    coss = []
    for refr, shr in ((reft_ref, sht_ref), (refs_ref, shs_ref)):
        refX = refr[...]
        shX = shr[...]
        ns2r = _dotT(ones_row, shX * shX)
        nr2 = jnp.sum(refX * refX, axis=1, keepdims=True)
        G_rs1 = _dotT(refX, shX)
        G_sh = _dotT(high, shX)
        G_rf = _dotT(high, refX)
        G_rh = jnp.sum(jnp.where(diag, G_rf, 0.0), axis=1, keepdims=True)
        G_rs = jnp.concatenate([G_rs1] * _K, axis=0)
        nr2_kr = jnp.concatenate([nr2] * _K, axis=0)

        d_sr2 = ns2r - 2.0 * G_rs + nr2_kr
        d_hr2 = nh2 - 2.0 * G_rh + nr2_kr
        d_sh2 = ns2r - 2.0 * G_sh + nh2
        inv_sr = _inv_clamped(d_sr2)
        inv_hr = _inv_clamped(d_hr2)
        inv_sh = _inv_clamped(d_sh2)

        c1 = (G_sh - G_rs - G_rh + nr2_kr) * inv_sr * inv_hr
        c2 = (G_rs - G_rh - G_sh + nh2) * inv_hr * inv_sh
        c3 = (G_rh - G_sh - G_rs + ns2r) * inv_sr * inv_sh
        coss.append((c1, c2, c3))

    contrib = jnp.zeros((1, 1), jnp.float32)
    for a in range(3):
        contrib = contrib + jnp.sum(jnp.abs(coss[1][a] - coss[0][a]),
                                    axis=(0, 1), keepdims=True)

    @pl.when((b == 0) & (j == 0))
    def _():
        out_ref[...] = jnp.zeros((1, 1), jnp.float32)

    out_ref[...] += contrib

    @pl.when((b == _B - 1) & (j == 2))
    def _():
        out_ref[...] = out_ref[...] / jnp.float32(_TOTAL)


def _angle_loss(gath_t, gath_s, high):
    return pl.pallas_call(
        _angle_body,
        grid=(_B, 3),
        in_specs=[
            pl.BlockSpec((_NR, _D), lambda b, j: (4 * b, 0)),
            pl.BlockSpec((_NR, _D), lambda b, j: (4 * b, 0)),
            pl.BlockSpec((_NR, _D), lambda b, j: (4 * b + 1 + j, 0)),
            pl.BlockSpec((_NR, _D), lambda b, j: (4 * b + 1 + j, 0)),
            pl.BlockSpec((4 * _NR, _D), lambda b, j: (b, 0)),
        ],
        out_specs=pl.BlockSpec((1, 1), lambda b, j: (0, 0)),
        out_shape=jax.ShapeDtypeStruct((1, 1), jnp.float32),
    )(gath_t, gath_s, gath_t, gath_s, high)


def kernel(teacher_feats, student_feats, ref_perm, shared_perm):
    tf = teacher_feats.reshape(_B * 8 * _P, _D)
    sf = student_feats.reshape(_B * 4 * _P, _D)
    rp = ref_perm.astype(jnp.int32)
    sp = shared_perm.astype(jnp.int32)

    t_rows, s_rows = [], []
    for b in range(_B):
        t_rows.append(b * (8 * _P) + rp)
        s_rows.append(b * (4 * _P) + rp)
        for tfi, sfi in _PAIRS:
            t_rows.append(b * (8 * _P) + tfi * _P + sp)
            s_rows.append(b * (4 * _P) + sfi * _P + sp)
    idx_t = jnp.concatenate(t_rows)
    idx_s = jnp.concatenate(s_rows)

    gath_t = _sc_gather_rows(tf, idx_t, 1024)
    gath_s = _sc_gather_rows(sf, idx_s, 1024)
    gidx = _sim_topk(gath_t, teacher_feats)
    high = _sc_gather_rows(tf, gidx.reshape(1024), 1024)
    loss = _angle_loss(gath_t, gath_s, high)
    return loss.reshape(())


# speedup vs baseline: 62.5166x; 1.0860x over previous
import functools

import jax
import jax.numpy as jnp
from jax import lax
from jax.experimental import pallas as pl
from jax.experimental.pallas import tpu as pltpu
from jax.experimental.pallas import tpu_sc as plsc

_B, _P, _D = 2, 1024, 1024
_NR, _NS, _K = 128, 128, 4
_PAIRS = ((2, 1), (4, 2), (6, 3))
_TOTAL = 3 * _B * _NR * _NS * _K


def _dotT(a, b):
    return lax.dot_general(a, b, (((1,), (1,)), ((), ())),
                           preferred_element_type=jnp.float32,
                           precision=lax.Precision.DEFAULT)


def _sc_gather_rows(table, idx, n_rows):
    n_workers = 32
    per_w = n_rows // n_workers
    mesh = plsc.VectorSubcoreMesh(core_axis_name="c", subcore_axis_name="s")

    @functools.partial(
        pl.kernel,
        mesh=mesh,
        out_type=jax.ShapeDtypeStruct((n_rows, _D), jnp.float32),
        scratch_types=[
            pltpu.VMEM((per_w,), jnp.int32),
            pltpu.VMEM((per_w, _D), jnp.float32),
            pltpu.SemaphoreType.DMA,
        ],
    )
    def k(table_hbm, idx_hbm, out_hbm, idx_v, rows_v, sem):
        wid = lax.axis_index("s") * 2 + lax.axis_index("c")
        base = wid * per_w
        pltpu.sync_copy(idx_hbm.at[pl.ds(base, per_w)], idx_v)
        pltpu.async_copy(table_hbm.at[idx_v], rows_v, sem).wait()
        pltpu.sync_copy(rows_v, out_hbm.at[pl.ds(base, per_w)])

    return k(table, idx)


def _sc_gather_rows2(table_a, idx_a, table_b, idx_b, n_rows):
    n_workers = 32
    per_w = n_rows // n_workers
    mesh = plsc.VectorSubcoreMesh(core_axis_name="c", subcore_axis_name="s")

    @functools.partial(
        pl.kernel,
        mesh=mesh,
        out_type=(jax.ShapeDtypeStruct((n_rows, _D), jnp.float32),
                  jax.ShapeDtypeStruct((n_rows, _D), jnp.float32)),
        scratch_types=[
            pltpu.VMEM((per_w,), jnp.int32),
            pltpu.VMEM((per_w, _D), jnp.float32),
            pltpu.SemaphoreType.DMA,
        ],
    )
    def k(ta_hbm, ia_hbm, tb_hbm, ib_hbm, oa_hbm, ob_hbm, idx_v, rows_v, sem):
        wid = lax.axis_index("s") * 2 + lax.axis_index("c")
        base = wid * per_w
        pltpu.sync_copy(ia_hbm.at[pl.ds(base, per_w)], idx_v)
        pltpu.async_copy(ta_hbm.at[idx_v], rows_v, sem).wait()
        pltpu.sync_copy(rows_v, oa_hbm.at[pl.ds(base, per_w)])
        pltpu.sync_copy(ib_hbm.at[pl.ds(base, per_w)], idx_v)
        pltpu.async_copy(tb_hbm.at[idx_v], rows_v, sem).wait()
        pltpu.sync_copy(rows_v, ob_hbm.at[pl.ds(base, per_w)])

    return k(table_a, idx_a, table_b, idx_b)


def _sim_topk_body(reft_ref, frame_ref, gidx_ref, sim_ref):
    b = pl.program_id(0)
    f = pl.program_id(1)
    frame = frame_ref[0, 0]
    reft = reft_ref[...]
    nr2 = jnp.sum(reft * reft, axis=1, keepdims=True)
    rtn = reft * jnp.where(nr2 > 1e-24, lax.rsqrt(jnp.maximum(nr2, 1e-30)),
                           jnp.float32(1e12))
    s = _dotT(frame, rtn)
    ne2 = jnp.sum(frame * frame, axis=1, keepdims=True)
    sim_ref[f] = s * jnp.where(ne2 > 1e-24, lax.rsqrt(jnp.maximum(ne2, 1e-30)),
                               jnp.float32(1e12))

    @pl.when(f == 3)
    def _():
        work = sim_ref[...].reshape(4 * _P, _NR)
        row_iota = lax.broadcasted_iota(jnp.int32, (4 * _P, _NR), 0)
        gs = []
        for _k in range(_K):
            m = jnp.max(work, axis=0, keepdims=True)
            idx = jnp.min(jnp.where(work == m, row_iota, 4 * _P),
                          axis=0, keepdims=True)
            fi = idx // _P
            p = idx - fi * _P
            gs.append(b * (8 * _P) + fi * (2 * _P) + _P + p)
            work = jnp.where(row_iota == idx, -jnp.inf, work)
        gidx_ref[0] = jnp.concatenate(gs, axis=0)


def _sim_topk(gath_t, teacher):
    return pl.pallas_call(
        _sim_topk_body,
        grid=(_B, 4),
        in_specs=[
            pl.BlockSpec((_NR, _D), lambda b, f: (b, 0)),
            pl.BlockSpec((1, 1, _P, _D), lambda b, f: (b, 2 * f + 1, 0, 0)),
        ],
        out_specs=pl.BlockSpec((1, _K, _NR), lambda b, f: (b, 0, 0)),
        out_shape=jax.ShapeDtypeStruct((_B, _K, _NR), jnp.int32),
        scratch_shapes=[pltpu.VMEM((4, _P, _NR), jnp.float32)],
    )(gath_t, teacher)


def _angle_body(reft_ref, refs_ref, sht_ref, shs_ref, high_ref, out_ref):
    b = pl.program_id(0)
    j = pl.program_id(1)
    high = high_ref[...]
    ones_row = jnp.ones((1, _D), jnp.float32)

    def _inv_clamped(d2):
        return jnp.where(d2 > 1e-16, lax.rsqrt(jnp.maximum(d2, 1e-30)),
                         jnp.float32(1e8))

    nh2 = _dotT(high * high, ones_row)
    rows = lax.broadcasted_iota(jnp.int32, (4 * _NR, _NR), 0)
    cols = lax.broadcasted_iota(jnp.int32, (4 * _NR, _NR), 1)
    diag = rows % _NR == cols

    coss = []
    for refr, shr in ((reft_ref, sht_ref), (refs_ref, shs_ref)):
        refX = refr[...]
        shX = shr[...]
        ns2r = _dotT(ones_row, shX * shX)
        nr2 = jnp.sum(refX * refX, axis=1, keepdims=True)
        G_rs1 = _dotT(refX, shX)
        G_sh = _dotT(high, shX)
        G_rf = _dotT(high, refX)
        G_rh = jnp.sum(jnp.where(diag, G_rf, 0.0), axis=1, keepdims=True)
        G_rs = jnp.concatenate([G_rs1] * _K, axis=0)
        nr2_kr = jnp.concatenate([nr2] * _K, axis=0)

        d_sr2 = ns2r - 2.0 * G_rs + nr2_kr
        d_hr2 = nh2 - 2.0 * G_rh + nr2_kr
        d_sh2 = ns2r - 2.0 * G_sh + nh2
        inv_sr = _inv_clamped(d_sr2)
        inv_hr = _inv_clamped(d_hr2)
        inv_sh = _inv_clamped(d_sh2)

        c1 = (G_sh - G_rs - G_rh + nr2_kr) * inv_sr * inv_hr
        c2 = (G_rs - G_rh - G_sh + nh2) * inv_hr * inv_sh
        c3 = (G_rh - G_sh - G_rs + ns2r) * inv_sr * inv_sh
        coss.append((c1, c2, c3))

    contrib = jnp.zeros((1, 1), jnp.float32)
    for a in range(3):
        contrib = contrib + jnp.sum(jnp.abs(coss[1][a] - coss[0][a]),
                                    axis=(0, 1), keepdims=True)

    @pl.when((b == 0) & (j == 0))
    def _():
        out_ref[...] = jnp.zeros((1, 1), jnp.float32)

    out_ref[...] += contrib

    @pl.when((b == _B - 1) & (j == 2))
    def _():
        out_ref[...] = out_ref[...] / jnp.float32(_TOTAL)


def _angle_loss(ref_t, ref_s, sh_t, sh_s, high):
    return pl.pallas_call(
        _angle_body,
        grid=(_B, 3),
        in_specs=[
            pl.BlockSpec((_NR, _D), lambda b, j: (b, 0)),
            pl.BlockSpec((_NR, _D), lambda b, j: (b, 0)),
            pl.BlockSpec((_NR, _D), lambda b, j: (3 * b + j, 0)),
            pl.BlockSpec((_NR, _D), lambda b, j: (3 * b + j, 0)),
            pl.BlockSpec((4 * _NR, _D), lambda b, j: (b, 0)),
        ],
        out_specs=pl.BlockSpec((1, 1), lambda b, j: (0, 0)),
        out_shape=jax.ShapeDtypeStruct((1, 1), jnp.float32),
    )(ref_t, ref_s, sh_t, sh_s, high)


def kernel(teacher_feats, student_feats, ref_perm, shared_perm):
    tf = teacher_feats.reshape(_B * 8 * _P, _D)
    sf = student_feats.reshape(_B * 4 * _P, _D)
    rp = ref_perm.astype(jnp.int32)
    sp = shared_perm.astype(jnp.int32)

    rt_idx = jnp.concatenate([b * (8 * _P) + rp for b in range(_B)])
    rs_idx = jnp.concatenate([b * (4 * _P) + rp for b in range(_B)])
    sht_idx = jnp.concatenate([b * (8 * _P) + tfi * _P + sp
                               for b in range(_B) for tfi, _ in _PAIRS])
    shs_idx = jnp.concatenate([b * (4 * _P) + sfi * _P + sp
                               for b in range(_B) for _, sfi in _PAIRS])

    ref_t, ref_s = _sc_gather_rows2(tf, rt_idx, sf, rs_idx, _B * _NR)
    sh_t, sh_s = _sc_gather_rows2(tf, sht_idx, sf, shs_idx, _B * 3 * _NR)
    gidx = _sim_topk(ref_t, teacher_feats)
    high = _sc_gather_rows(tf, gidx.reshape(1024), 1024)
    loss = _angle_loss(ref_t, ref_s, sh_t, sh_s, high)
    return loss.reshape(())
